# Initial kernel scaffold; baseline (speedup 1.0000x reference)
import jax
import jax.numpy as jnp
from jax.experimental import pallas as pl


def kernel(E_V, E_E, edge_proj_w, edge_proj_b, w_in_w, w_in_b, graph_token, enc_qkv_w, enc_qkv_b, enc_ao_w, enc_ao_b, enc_ln1_g, enc_ln1_b, enc_i_w, enc_i_b, enc_o_w, enc_o_b, enc_ln2_g, enc_ln2_b, x, edge_attr, id_key):
    raise NotImplementedError("write your pallas kernel here")


# speedup vs baseline: 2.2287x; 2.2287x over previous
import functools

import jax
import jax.numpy as jnp
from jax.experimental import pallas as pl
from jax.experimental.pallas import tpu as pltpu

G = 128
NN = 32
NE = 64
DP = 64
DE = 64
D = 512
H = 8
DH = D // H
L = 8
SV = 1 + NN + NE
S = 112
EPS = 1e-12

_GELU_C = 0.7978845608028654


def _gelu_tanh(x):
    return 0.5 * x * (1.0 + jnp.tanh(_GELU_C * (x + 0.044715 * x * x * x)))


def _layernorm(y, g, b, eps):
    mean = jnp.mean(y, axis=-1, keepdims=True)
    c = y - mean
    var = jnp.mean(c * c, axis=-1, keepdims=True)
    return c * jax.lax.rsqrt(var + eps) * g + b


def _proj_kernel(x_ref, w_ref, b_ref, o_ref):
    o_ref[...] = (jnp.dot(x_ref[...], w_ref[...],
                          preferred_element_type=jnp.float32)
                  + b_ref[...]).astype(o_ref.dtype)


def _input_proj(x_bf16, w_bf16, b2):
    M, K = x_bf16.shape
    N = w_bf16.shape[1]
    tm = 1024
    return pl.pallas_call(
        _proj_kernel,
        out_shape=jax.ShapeDtypeStruct((M, N), jnp.float32),
        grid=(M // tm,),
        in_specs=[
            pl.BlockSpec((tm, K), lambda i: (i, 0)),
            pl.BlockSpec((K, N), lambda i: (0, 0)),
            pl.BlockSpec((1, N), lambda i: (0, 0)),
        ],
        out_specs=pl.BlockSpec((tm, N), lambda i: (i, 0)),
        compiler_params=pltpu.CompilerParams(
            dimension_semantics=("parallel",)),
    )(x_bf16, w_bf16, b2)


def _enc_kernel(h_ref, qw_ref, qb_ref, aw_ref, ab_ref, g1_ref, b1_ref,
                iw_ref, ib_ref, ow_ref, ob_ref, g2_ref, b2_ref,
                o_ref, h_scr, ctx_scr, *, bt, cg):
    l = pl.program_id(1)
    R = bt * S

    @pl.when(l == 0)
    def _():
        h_scr[...] = h_ref[...].reshape(R, D)

    x = h_scr[...]
    xb = x.astype(jnp.bfloat16)
    qkv = (jnp.dot(xb, qw_ref[0], preferred_element_type=jnp.float32)
           + qb_ref[0])
    qkvb = qkv.astype(jnp.bfloat16)

    RC = cg * S
    rg = jax.lax.broadcasted_iota(jnp.int32, (RC, RC), 0) // S
    cc = jax.lax.broadcasted_iota(jnp.int32, (RC, RC), 1)
    bias = jnp.where((rg == cc // S) & (cc % S < SV), 0.0, -1e9)

    for c in range(bt // cg):
        r0 = c * RC
        for h in range(H):
            q = qkvb[r0:r0 + RC, h * DH:(h + 1) * DH]
            k = qkvb[r0:r0 + RC, D + h * DH:D + (h + 1) * DH]
            v = qkvb[r0:r0 + RC, 2 * D + h * DH:2 * D + (h + 1) * DH]
            s = jax.lax.dot_general(q, k, (((1,), (1,)), ((), ())),
                                    preferred_element_type=jnp.float32)
            s = s + bias
            m = jnp.max(s, axis=-1, keepdims=True)
            p = jnp.exp(s - m)
            p = p * pl.reciprocal(jnp.sum(p, axis=-1, keepdims=True),
                                  approx=True)
            ctx = jnp.dot(p.astype(jnp.bfloat16), v,
                          preferred_element_type=jnp.float32)
            ctx_scr[r0:r0 + RC, h * DH:(h + 1) * DH] = ctx

    attn = (jnp.dot(ctx_scr[...].astype(jnp.bfloat16), aw_ref[0],
                    preferred_element_type=jnp.float32) + ab_ref[0])
    h1 = _layernorm(attn + x, g1_ref[0], b1_ref[0], EPS)

    inter = (jnp.dot(h1.astype(jnp.bfloat16), iw_ref[0],
                     preferred_element_type=jnp.float32) + ib_ref[0])
    inter = _gelu_tanh(inter)
    ffn = (jnp.dot(inter.astype(jnp.bfloat16), ow_ref[0],
                   preferred_element_type=jnp.float32) + ob_ref[0])
    h2 = _layernorm(ffn + h1, g2_ref[0], b2_ref[0], EPS)
    h_scr[...] = h2

    @pl.when(l == pl.num_programs(1) - 1)
    def _():
        o_ref[...] = h2.reshape(bt, S, D).astype(o_ref.dtype)


def _encoder(tokens, stk, *, bt=8, cg=4):
    kern = functools.partial(_enc_kernel, bt=bt, cg=cg)

    def wspec(shape):
        n = len(shape)
        return pl.BlockSpec((1,) + shape, lambda b, l: (l,) + (0,) * n)

    return pl.pallas_call(
        kern,
        out_shape=jax.ShapeDtypeStruct((G, S, D), jnp.float32),
        grid=(G // bt, L),
        in_specs=[
            pl.BlockSpec((bt, S, D), lambda b, l: (b, 0, 0)),
            wspec((D, 3 * D)), wspec((1, 3 * D)),
            wspec((D, D)), wspec((1, D)),
            wspec((1, D)), wspec((1, D)),
            wspec((D, 2 * D)), wspec((1, 2 * D)),
            wspec((2 * D, D)), wspec((1, D)),
            wspec((1, D)), wspec((1, D)),
        ],
        out_specs=pl.BlockSpec((bt, S, D), lambda b, l: (b, 0, 0)),
        scratch_shapes=[pltpu.VMEM((bt * S, D), jnp.float32),
                        pltpu.VMEM((bt * S, D), jnp.float32)],
        compiler_params=pltpu.CompilerParams(
            dimension_semantics=("parallel", "arbitrary"),
            vmem_limit_bytes=50 * 1024 * 1024),
    )(tokens,
      stk["qkv_w"], stk["qkv_b"], stk["ao_w"], stk["ao_b"],
      stk["ln1_g"], stk["ln1_b"], stk["i_w"], stk["i_b"],
      stk["o_w"], stk["o_b"], stk["ln2_g"], stk["ln2_b"])


def kernel(E_V, E_E, edge_proj_w, edge_proj_b, w_in_w, w_in_b, graph_token,
           enc_qkv_w, enc_qkv_b, enc_ao_w, enc_ao_b, enc_ln1_g, enc_ln1_b,
           enc_i_w, enc_i_b, enc_o_w, enc_o_b, enc_ln2_g, enc_ln2_b,
           x, edge_attr, id_key):
    key = id_key
    kgs = []
    for _ in range(G):
        key, kg, _kp = jax.random.split(key, 3)
        kgs.append(kg)
    kg_stack = jnp.stack(kgs)
    gm = jax.vmap(lambda k: jax.random.normal(k, (NN, NN), jnp.float32))(
        kg_stack)
    q_orf, _ = jnp.linalg.qr(gm)
    P3 = jnp.pad(q_orf, ((0, 0), (0, 0), (0, DP - NN)))
    P = P3.reshape(G * NN, DP)

    ea = (edge_attr.astype(jnp.float32) @ edge_proj_w + edge_proj_b)
    P_src = jnp.tile(P3, (1, 2, 1)).reshape(G * NE, DP)
    P_dst = jnp.tile(jnp.roll(P3, -1, axis=1), (1, 2, 1)).reshape(G * NE, DP)
    X_v = jnp.concatenate(
        [x, P, P, jnp.broadcast_to(E_V, (G * NN, DE))], axis=1)
    X_e = jnp.concatenate(
        [ea, P_src, P_dst, jnp.broadcast_to(E_E, (G * NE, DE))], axis=1)
    X_all = jnp.concatenate([X_v, X_e], axis=0).astype(jnp.bfloat16)

    Xp = _input_proj(X_all, w_in_w, w_in_b)
    Xv = Xp[:G * NN].reshape(G, NN, D)
    Xe = Xp[G * NN:].reshape(G, NE, D)
    gt = jnp.broadcast_to(graph_token.reshape(1, 1, D), (G, 1, D))
    pad = jnp.zeros((G, S - SV, D), jnp.float32)
    tokens = jnp.concatenate([gt, Xv, Xe, pad], axis=1)

    stk = {"qkv_w": enc_qkv_w, "qkv_b": enc_qkv_b,
           "ao_w": enc_ao_w, "ao_b": enc_ao_b,
           "ln1_g": enc_ln1_g, "ln1_b": enc_ln1_b,
           "i_w": enc_i_w, "i_b": enc_i_b,
           "o_w": enc_o_w, "o_b": enc_o_b,
           "ln2_g": enc_ln2_g, "ln2_b": enc_ln2_b}
    h = _encoder(tokens, stk)

    masks = jnp.ones((G, SV), dtype=bool)
    return h[:, :SV], masks


# speedup vs baseline: 2.3520x; 1.0553x over previous
import functools

import jax
import jax.numpy as jnp
from jax.experimental import pallas as pl
from jax.experimental.pallas import tpu as pltpu

G = 128
NN = 32
NE = 64
DP = 64
DE = 64
D = 512
H = 8
DH = D // H
L = 8
SV = 1 + NN + NE
S = 104
EPS = 1e-12

_GELU_C = 0.7978845608028654


def _gelu_tanh(x):
    return 0.5 * x * (1.0 + jnp.tanh(_GELU_C * (x + 0.044715 * x * x * x)))


def _layernorm(y, g, b, eps):
    mean = jnp.mean(y, axis=-1, keepdims=True)
    c = y - mean
    var = jnp.mean(c * c, axis=-1, keepdims=True)
    return c * jax.lax.rsqrt(var + eps) * g + b


def _proj_kernel(x_ref, w_ref, b_ref, o_ref):
    o_ref[...] = (jnp.dot(x_ref[...], w_ref[...],
                          preferred_element_type=jnp.float32)
                  + b_ref[...]).astype(o_ref.dtype)


def _input_proj(x_bf16, w_bf16, b2):
    M, K = x_bf16.shape
    N = w_bf16.shape[1]
    tm = 1024
    return pl.pallas_call(
        _proj_kernel,
        out_shape=jax.ShapeDtypeStruct((M, N), jnp.float32),
        grid=(M // tm,),
        in_specs=[
            pl.BlockSpec((tm, K), lambda i: (i, 0)),
            pl.BlockSpec((K, N), lambda i: (0, 0)),
            pl.BlockSpec((1, N), lambda i: (0, 0)),
        ],
        out_specs=pl.BlockSpec((tm, N), lambda i: (i, 0)),
        compiler_params=pltpu.CompilerParams(
            dimension_semantics=("parallel",)),
    )(x_bf16, w_bf16, b2)


def _enc_kernel(h_ref, qw_ref, qb_ref, aw_ref, ab_ref, g1_ref, b1_ref,
                iw_ref, ib_ref, ow_ref, ob_ref, g2_ref, b2_ref,
                o_ref, h_scr, ctx_scr, *, bt, cg):
    l = pl.program_id(1)
    R = bt * S

    @pl.when(l == 0)
    def _():
        h_scr[...] = h_ref[...].reshape(R, D)

    x = h_scr[...]
    xb = x.astype(jnp.bfloat16)
    qkv = (jnp.dot(xb, qw_ref[0], preferred_element_type=jnp.float32)
           + qb_ref[0])
    qkvb = qkv.astype(jnp.bfloat16)

    RC = cg * S
    rg = jax.lax.broadcasted_iota(jnp.int32, (RC, RC), 0) // S
    cc = jax.lax.broadcasted_iota(jnp.int32, (RC, RC), 1)
    bias = jnp.where((rg == cc // S) & (cc % S < SV), 0.0, -1e9)

    for c in range(bt // cg):
        r0 = c * RC
        for h in range(H):
            q = qkvb[r0:r0 + RC, h * DH:(h + 1) * DH]
            k = qkvb[r0:r0 + RC, D + h * DH:D + (h + 1) * DH]
            v = qkvb[r0:r0 + RC, 2 * D + h * DH:2 * D + (h + 1) * DH]
            s = jax.lax.dot_general(q, k, (((1,), (1,)), ((), ())),
                                    preferred_element_type=jnp.float32)
            s = s + bias
            m = jnp.max(s, axis=-1, keepdims=True)
            p = jnp.exp(s - m)
            p = p * pl.reciprocal(jnp.sum(p, axis=-1, keepdims=True),
                                  approx=True)
            ctx = jnp.dot(p.astype(jnp.bfloat16), v,
                          preferred_element_type=jnp.float32)
            ctx_scr[r0:r0 + RC, h * DH:(h + 1) * DH] = ctx

    attn = (jnp.dot(ctx_scr[...].astype(jnp.bfloat16), aw_ref[0],
                    preferred_element_type=jnp.float32) + ab_ref[0])
    h1 = _layernorm(attn + x, g1_ref[0], b1_ref[0], EPS)

    inter = (jnp.dot(h1.astype(jnp.bfloat16), iw_ref[0],
                     preferred_element_type=jnp.float32) + ib_ref[0])
    inter = _gelu_tanh(inter)
    ffn = (jnp.dot(inter.astype(jnp.bfloat16), ow_ref[0],
                   preferred_element_type=jnp.float32) + ob_ref[0])
    h2 = _layernorm(ffn + h1, g2_ref[0], b2_ref[0], EPS)
    h_scr[...] = h2

    @pl.when(l == pl.num_programs(1) - 1)
    def _():
        o_ref[...] = h2.reshape(bt, S, D)[:, :SV, :].astype(o_ref.dtype)


def _encoder(tokens, stk, *, bt=8, cg=2):
    kern = functools.partial(_enc_kernel, bt=bt, cg=cg)

    def wspec(shape):
        n = len(shape)
        return pl.BlockSpec((1,) + shape, lambda b, l: (l,) + (0,) * n)

    return pl.pallas_call(
        kern,
        out_shape=jax.ShapeDtypeStruct((G, SV, D), jnp.float32),
        grid=(G // bt, L),
        in_specs=[
            pl.BlockSpec((bt, S, D), lambda b, l: (b, 0, 0)),
            wspec((D, 3 * D)), wspec((1, 3 * D)),
            wspec((D, D)), wspec((1, D)),
            wspec((1, D)), wspec((1, D)),
            wspec((D, 2 * D)), wspec((1, 2 * D)),
            wspec((2 * D, D)), wspec((1, D)),
            wspec((1, D)), wspec((1, D)),
        ],
        out_specs=pl.BlockSpec((bt, SV, D), lambda b, l: (b, 0, 0)),
        scratch_shapes=[pltpu.VMEM((bt * S, D), jnp.float32),
                        pltpu.VMEM((bt * S, D), jnp.float32)],
        compiler_params=pltpu.CompilerParams(
            dimension_semantics=("parallel", "arbitrary"),
            vmem_limit_bytes=50 * 1024 * 1024),
    )(tokens,
      stk["qkv_w"], stk["qkv_b"], stk["ao_w"], stk["ao_b"],
      stk["ln1_g"], stk["ln1_b"], stk["i_w"], stk["i_b"],
      stk["o_w"], stk["o_b"], stk["ln2_g"], stk["ln2_b"])


def kernel(E_V, E_E, edge_proj_w, edge_proj_b, w_in_w, w_in_b, graph_token,
           enc_qkv_w, enc_qkv_b, enc_ao_w, enc_ao_b, enc_ln1_g, enc_ln1_b,
           enc_i_w, enc_i_b, enc_o_w, enc_o_b, enc_ln2_g, enc_ln2_b,
           x, edge_attr, id_key):
    key = id_key
    kgs = []
    for _ in range(G):
        key, kg, _kp = jax.random.split(key, 3)
        kgs.append(kg)
    kg_stack = jnp.stack(kgs)
    gm = jax.vmap(lambda k: jax.random.normal(k, (NN, NN), jnp.float32))(
        kg_stack)
    q_orf, _ = jnp.linalg.qr(gm)
    P3 = jnp.pad(q_orf, ((0, 0), (0, 0), (0, DP - NN)))
    P = P3.reshape(G * NN, DP)

    ea = (edge_attr.astype(jnp.float32) @ edge_proj_w + edge_proj_b)
    P_src = jnp.tile(P3, (1, 2, 1)).reshape(G * NE, DP)
    P_dst = jnp.tile(jnp.roll(P3, -1, axis=1), (1, 2, 1)).reshape(G * NE, DP)
    X_v = jnp.concatenate(
        [x, P, P, jnp.broadcast_to(E_V, (G * NN, DE))], axis=1)
    X_e = jnp.concatenate(
        [ea, P_src, P_dst, jnp.broadcast_to(E_E, (G * NE, DE))], axis=1)
    X_all = jnp.concatenate([X_v, X_e], axis=0).astype(jnp.bfloat16)

    Xp = _input_proj(X_all, w_in_w, w_in_b)
    Xv = Xp[:G * NN].reshape(G, NN, D)
    Xe = Xp[G * NN:].reshape(G, NE, D)
    gt = jnp.broadcast_to(graph_token.reshape(1, 1, D), (G, 1, D))
    pad = jnp.zeros((G, S - SV, D), jnp.float32)
    tokens = jnp.concatenate([gt, Xv, Xe, pad], axis=1)

    stk = {"qkv_w": enc_qkv_w, "qkv_b": enc_qkv_b,
           "ao_w": enc_ao_w, "ao_b": enc_ao_b,
           "ln1_g": enc_ln1_g, "ln1_b": enc_ln1_b,
           "i_w": enc_i_w, "i_b": enc_i_b,
           "o_w": enc_o_w, "o_b": enc_o_b,
           "ln2_g": enc_ln2_g, "ln2_b": enc_ln2_b}
    h = _encoder(tokens, stk)

    masks = jnp.ones((G, SV), dtype=bool)
    return h, masks


# speedup vs baseline: 3.2745x; 1.3922x over previous
import functools

import jax
import jax.numpy as jnp
import numpy as np
from jax.experimental import pallas as pl
from jax.experimental.pallas import tpu as pltpu

G = 128
NN = 32
NE = 64
DP = 64
DE = 64
FD = 256
D = 512
H = 8
DH = D // H
L = 8
SV = 1 + NN + NE
S = 104
EPS = 1e-12

_GELU_C = 0.7978845608028654


def _gelu_tanh(x):
    return 0.5 * x * (1.0 + jnp.tanh(_GELU_C * (x + 0.044715 * x * x * x)))


def _layernorm(y, g, b, eps):
    mean = jnp.mean(y, axis=-1, keepdims=True)
    c = y - mean
    var = jnp.mean(c * c, axis=-1, keepdims=True)
    return c * jax.lax.rsqrt(var + eps) * g + b


_ROT1 = (13, 15, 26, 6)
_ROT2 = (17, 29, 16, 24)
_KS_PARITY = 0x1BD11BDA


def _tf_hash(k0, k1, c0, c1):
    ks2 = k0 ^ k1 ^ _KS_PARITY

    def rot(v, d):
        return jax.lax.shift_left(v, np.int32(d)) | jax.lax.shift_right_logical(
            v, np.int32(32 - d))

    def rounds(x0, x1, rots):
        for r in rots:
            x0 = x0 + x1
            x1 = rot(x1, r)
            x1 = x0 ^ x1
        return x0, x1

    x0, x1 = c0 + k0, c1 + k1
    x0, x1 = rounds(x0, x1, _ROT1)
    x0, x1 = x0 + k1, x1 + ks2 + 1
    x0, x1 = rounds(x0, x1, _ROT2)
    x0, x1 = x0 + ks2, x1 + k0 + 2
    x0, x1 = rounds(x0, x1, _ROT1)
    x0, x1 = x0 + k0, x1 + k1 + 3
    x0, x1 = rounds(x0, x1, _ROT2)
    x0, x1 = x0 + k1, x1 + ks2 + 4
    x0, x1 = rounds(x0, x1, _ROT1)
    x0, x1 = x0 + ks2, x1 + k0 + 5
    return x0, x1


def _chain_kernel(key_ref, out_ref):
    zero = jnp.int32(0)
    one = jnp.int32(1)

    def body(i, carry):
        k0, k1 = carry
        g0, g1 = _tf_hash(k0, k1, zero, one)
        out_ref[2 * i] = g0
        out_ref[2 * i + 1] = g1
        return _tf_hash(k0, k1, zero, zero)

    k = jax.lax.fori_loop(0, G, body, (key_ref[0], key_ref[1]))
    out_ref[0] = out_ref[0] + zero * k[0]


def _split_chain(id_key):
    key_i32 = jax.lax.bitcast_convert_type(id_key, jnp.int32)
    out = pl.pallas_call(
        _chain_kernel,
        out_shape=jax.ShapeDtypeStruct((2 * G,), jnp.int32),
        in_specs=[pl.BlockSpec(memory_space=pltpu.SMEM)],
        out_specs=pl.BlockSpec(memory_space=pltpu.SMEM),
    )(key_i32)
    return jax.lax.bitcast_convert_type(out.reshape(G, 2), jnp.uint32)


def _enc_kernel(x_ref, ea_ref, p_ref, epw_ref, epb_ref, ev_ref, ee_ref,
                win_ref, winb_ref, gt_ref,
                qw_ref, qb_ref, aw_ref, ab_ref, g1_ref, b1_ref,
                iw_ref, ib_ref, ow_ref, ob_ref, g2_ref, b2_ref,
                o_ref, h_scr, ctx_scr, *, bt):
    l = pl.program_id(1)
    R = bt * S

    @pl.when(l == 0)
    def _():
        win = win_ref[...]
        xb = x_ref[...].astype(jnp.bfloat16)
        p3 = p_ref[...]
        pb = p3.reshape(bt * NN, DP).astype(jnp.bfloat16)
        evb = jnp.broadcast_to(ev_ref[...], (bt * NN, DE)).astype(jnp.bfloat16)
        lhs_v = jnp.concatenate([xb, pb, pb, evb], axis=1)
        pv = (jnp.dot(lhs_v, win, preferred_element_type=jnp.float32)
              + winb_ref[...])
        eat = ea_ref[...]
        epw = epw_ref[...]
        ea = (eat[:, 0:1] * epw[0:1, :] + eat[:, 1:2] * epw[1:2, :]
              + eat[:, 2:3] * epw[2:3, :]) + epb_ref[...]
        rolled = jnp.concatenate([p3[:, 1:, :], p3[:, :1, :]], axis=1)
        psrc = jnp.concatenate([p3, p3], axis=1).reshape(bt * NE, DP)
        pdst = jnp.concatenate([rolled, rolled], axis=1).reshape(bt * NE, DP)
        eeb = jnp.broadcast_to(ee_ref[...], (bt * NE, DE))
        lhs_e = jnp.concatenate([ea, psrc, pdst, eeb],
                                axis=1).astype(jnp.bfloat16)
        pe = (jnp.dot(lhs_e, win, preferred_element_type=jnp.float32)
              + winb_ref[...])
        gt3 = jnp.broadcast_to(gt_ref[...].reshape(1, 1, D), (bt, 1, D))
        toks = jnp.concatenate(
            [gt3, pv.reshape(bt, NN, D), pe.reshape(bt, NE, D),
             jnp.zeros((bt, S - SV, D), jnp.float32)], axis=1)
        h_scr[...] = toks.reshape(R, D)

    x = h_scr[...]
    xb16 = x.astype(jnp.bfloat16)
    qkv = (jnp.dot(xb16, qw_ref[0], preferred_element_type=jnp.float32)
           + qb_ref[0])
    qkvb = qkv.astype(jnp.bfloat16)

    bias = jnp.where(
        jax.lax.broadcasted_iota(jnp.int32, (S, S), 1) < SV, 0.0, -1e9)

    for c in range(bt):
        r0 = c * S
        qg = qkvb[r0:r0 + S, :]
        for h in range(H):
            q = qg[:, h * DH:(h + 1) * DH]
            k = qg[:, D + h * DH:D + (h + 1) * DH]
            v = qg[:, 2 * D + h * DH:2 * D + (h + 1) * DH]
            s = jax.lax.dot_general(q, k, (((1,), (1,)), ((), ())),
                                    preferred_element_type=jnp.float32)
            p = jnp.exp(s + bias)
            p = p * pl.reciprocal(jnp.sum(p, axis=-1, keepdims=True),
                                  approx=True)
            ctx = jnp.dot(p.astype(jnp.bfloat16), v,
                          preferred_element_type=jnp.float32)
            ctx_scr[r0:r0 + S, h * DH:(h + 1) * DH] = ctx

    attn = (jnp.dot(ctx_scr[...].astype(jnp.bfloat16), aw_ref[0],
                    preferred_element_type=jnp.float32) + ab_ref[0])
    h1 = _layernorm(attn + x, g1_ref[0], b1_ref[0], EPS)

    inter = (jnp.dot(h1.astype(jnp.bfloat16), iw_ref[0],
                     preferred_element_type=jnp.float32) + ib_ref[0])
    inter = _gelu_tanh(inter)
    ffn = (jnp.dot(inter.astype(jnp.bfloat16), ow_ref[0],
                   preferred_element_type=jnp.float32) + ob_ref[0])
    h2 = _layernorm(ffn + h1, g2_ref[0], b2_ref[0], EPS)
    h_scr[...] = h2

    @pl.when(l == pl.num_programs(1) - 1)
    def _():
        o_ref[...] = h2.reshape(bt, S, D)[:, :SV, :].astype(o_ref.dtype)


def _encoder(x, edge_attr, P3, edge_proj_w, edge_proj_b, E_V, E_E,
             w_in_w, w_in_b, graph_token, stk, *, bt=8):
    kern = functools.partial(_enc_kernel, bt=bt)

    def wspec(shape):
        n = len(shape)
        return pl.BlockSpec((1,) + shape, lambda b, l: (l,) + (0,) * n)

    def cspec(shape):
        n = len(shape)
        return pl.BlockSpec(shape, lambda b, l: (0,) * n)

    return pl.pallas_call(
        kern,
        out_shape=jax.ShapeDtypeStruct((G, SV, D), jnp.float32),
        grid=(G // bt, L),
        in_specs=[
            pl.BlockSpec((bt * NN, FD), lambda b, l: (b, 0)),
            pl.BlockSpec((bt * NE, 3), lambda b, l: (b, 0)),
            pl.BlockSpec((bt, NN, DP), lambda b, l: (b, 0, 0)),
            cspec((3, FD)), cspec((1, FD)),
            cspec((1, DE)), cspec((1, DE)),
            cspec((FD + 2 * DP + DE, D)), cspec((1, D)),
            cspec((1, D)),
            wspec((D, 3 * D)), wspec((1, 3 * D)),
            wspec((D, D)), wspec((1, D)),
            wspec((1, D)), wspec((1, D)),
            wspec((D, 2 * D)), wspec((1, 2 * D)),
            wspec((2 * D, D)), wspec((1, D)),
            wspec((1, D)), wspec((1, D)),
        ],
        out_specs=pl.BlockSpec((bt, SV, D), lambda b, l: (b, 0, 0)),
        scratch_shapes=[pltpu.VMEM((bt * S, D), jnp.float32),
                        pltpu.VMEM((bt * S, D), jnp.float32)],
        compiler_params=pltpu.CompilerParams(
            dimension_semantics=("parallel", "arbitrary"),
            vmem_limit_bytes=50 * 1024 * 1024),
    )(x, edge_attr, P3,
      edge_proj_w, edge_proj_b, E_V, E_E, w_in_w, w_in_b, graph_token,
      stk["qkv_w"], stk["qkv_b"], stk["ao_w"], stk["ao_b"],
      stk["ln1_g"], stk["ln1_b"], stk["i_w"], stk["i_b"],
      stk["o_w"], stk["o_b"], stk["ln2_g"], stk["ln2_b"])


def kernel(E_V, E_E, edge_proj_w, edge_proj_b, w_in_w, w_in_b, graph_token,
           enc_qkv_w, enc_qkv_b, enc_ao_w, enc_ao_b, enc_ln1_g, enc_ln1_b,
           enc_i_w, enc_i_b, enc_o_w, enc_o_b, enc_ln2_g, enc_ln2_b,
           x, edge_attr, id_key):
    kg_stack = _split_chain(id_key)
    gm = jax.vmap(lambda k: jax.random.normal(k, (NN, NN), jnp.float32))(
        kg_stack)
    q_orf, _ = jnp.linalg.qr(gm)
    P3 = jnp.pad(q_orf, ((0, 0), (0, 0), (0, DP - NN)))

    stk = {"qkv_w": enc_qkv_w, "qkv_b": enc_qkv_b,
           "ao_w": enc_ao_w, "ao_b": enc_ao_b,
           "ln1_g": enc_ln1_g, "ln1_b": enc_ln1_b,
           "i_w": enc_i_w, "i_b": enc_i_b,
           "o_w": enc_o_w, "o_b": enc_o_b,
           "ln2_g": enc_ln2_g, "ln2_b": enc_ln2_b}
    h = _encoder(x, edge_attr, P3, edge_proj_w, edge_proj_b, E_V, E_E,
                 w_in_w, w_in_b, graph_token, stk)

    masks = jnp.ones((G, SV), dtype=bool)
    return h, masks


# speedup vs baseline: 3.8889x; 1.1877x over previous
import functools

import jax
import jax.numpy as jnp
import numpy as np
from jax.experimental import pallas as pl
from jax.experimental.pallas import tpu as pltpu

G = 128
NN = 32
NE = 64
DP = 64
DE = 64
FD = 256
D = 512
H = 8
DH = D // H
L = 8
SV = 1 + NN + NE
S = 104
EPS = 1e-12

_GELU_C = 0.7978845608028654


def _gelu_tanh(x):
    return 0.5 * x * (1.0 + jnp.tanh(_GELU_C * (x + 0.044715 * x * x * x)))


def _layernorm(y, g, b, eps):
    mean = jnp.mean(y, axis=-1, keepdims=True)
    c = y - mean
    var = jnp.mean(c * c, axis=-1, keepdims=True)
    return c * jax.lax.rsqrt(var + eps) * g + b


_ROT1 = (13, 15, 26, 6)
_ROT2 = (17, 29, 16, 24)
_KS_PARITY = 0x1BD11BDA


def _tf_hash(k0, k1, c0, c1):
    ks2 = k0 ^ k1 ^ _KS_PARITY

    def rot(v, d):
        return jax.lax.shift_left(v, np.int32(d)) | jax.lax.shift_right_logical(
            v, np.int32(32 - d))

    def rounds(x0, x1, rots):
        for r in rots:
            x0 = x0 + x1
            x1 = rot(x1, r)
            x1 = x0 ^ x1
        return x0, x1

    x0, x1 = c0 + k0, c1 + k1
    x0, x1 = rounds(x0, x1, _ROT1)
    x0, x1 = x0 + k1, x1 + ks2 + 1
    x0, x1 = rounds(x0, x1, _ROT2)
    x0, x1 = x0 + ks2, x1 + k0 + 2
    x0, x1 = rounds(x0, x1, _ROT1)
    x0, x1 = x0 + k0, x1 + k1 + 3
    x0, x1 = rounds(x0, x1, _ROT2)
    x0, x1 = x0 + k1, x1 + ks2 + 4
    x0, x1 = rounds(x0, x1, _ROT1)
    x0, x1 = x0 + ks2, x1 + k0 + 5
    return x0, x1


def _chain_kernel(key_ref, out_ref):
    zero = jnp.int32(0)
    one = jnp.int32(1)

    def body(i, carry):
        k0, k1 = carry
        g0, g1 = _tf_hash(k0, k1, zero, one)
        out_ref[2 * i] = g0
        out_ref[2 * i + 1] = g1
        return _tf_hash(k0, k1, zero, zero)

    k = jax.lax.fori_loop(0, G, body, (key_ref[0], key_ref[1]))
    out_ref[0] = out_ref[0] + zero * k[0]


def _split_chain(id_key):
    key_i32 = jax.lax.bitcast_convert_type(id_key, jnp.int32)
    out = pl.pallas_call(
        _chain_kernel,
        out_shape=jax.ShapeDtypeStruct((2 * G,), jnp.int32),
        in_specs=[pl.BlockSpec(memory_space=pltpu.SMEM)],
        out_specs=pl.BlockSpec(memory_space=pltpu.SMEM),
    )(key_i32)
    return jax.lax.bitcast_convert_type(out.reshape(G, 2), jnp.uint32)


def _qr_kernel(a_ref, o_ref, a_scr, v_scr, t_scr):
    a_scr[...] = a_ref[...]
    rowi = jax.lax.broadcasted_iota(jnp.int32, (NN, G), 0)
    for k in range(NN - 1):
        col = a_scr[:, k, :]
        alpha = col[k:k + 1, :]
        below = rowi > k
        xn2 = jnp.sum(jnp.where(below, col * col, 0.0), axis=0, keepdims=True)
        norm = jnp.sqrt(alpha * alpha + xn2)
        beta = jnp.where(alpha < 0, norm, -norm)
        tau = (beta - alpha) / beta
        scale = 1.0 / (alpha - beta)
        v = (jnp.where(below, col, 0.0) * scale
             + jnp.where(rowi == k, 1.0, 0.0))
        v_scr[:, k, :] = v
        t_scr[k:k + 1, :] = tau
        a = a_scr[...]
        w = jnp.sum(v[:, None, :] * a, axis=0)
        a_scr[...] = a - v[:, None, :] * (tau * w)[None, :, :]
    ri3 = jax.lax.broadcasted_iota(jnp.int32, (NN, NN, G), 0)
    ci3 = jax.lax.broadcasted_iota(jnp.int32, (NN, NN, G), 1)
    q = jnp.where(ri3 == ci3, 1.0, 0.0)
    for k in range(NN - 2, -1, -1):
        v = v_scr[:, k, :]
        tau = t_scr[k:k + 1, :]
        w = jnp.sum(v[:, None, :] * q, axis=0)
        q = q - v[:, None, :] * (tau * w)[None, :, :]
    o_ref[...] = q


def _qr_batched(gm):
    a_t = jnp.transpose(gm, (1, 2, 0))
    q_t = pl.pallas_call(
        _qr_kernel,
        out_shape=jax.ShapeDtypeStruct((NN, NN, G), jnp.float32),
        scratch_shapes=[pltpu.VMEM((NN, NN, G), jnp.float32),
                        pltpu.VMEM((NN, NN, G), jnp.float32),
                        pltpu.VMEM((NN, G), jnp.float32)],
    )(a_t)
    return jnp.transpose(q_t, (2, 0, 1))


def _enc_kernel(x_ref, ea_ref, p_ref, epw_ref, epb_ref, ev_ref, ee_ref,
                win_ref, winb_ref, gt_ref,
                qw_ref, qb_ref, aw_ref, ab_ref, g1_ref, b1_ref,
                iw_ref, ib_ref, ow_ref, ob_ref, g2_ref, b2_ref,
                o_ref, h_scr, ctx_scr, *, bt):
    l = pl.program_id(1)
    R = bt * S

    @pl.when(l == 0)
    def _():
        win = win_ref[...]
        xb = x_ref[...].astype(jnp.bfloat16)
        p3 = p_ref[...]
        pb = p3.reshape(bt * NN, DP).astype(jnp.bfloat16)
        evb = jnp.broadcast_to(ev_ref[...], (bt * NN, DE)).astype(jnp.bfloat16)
        lhs_v = jnp.concatenate([xb, pb, pb, evb], axis=1)
        pv = (jnp.dot(lhs_v, win, preferred_element_type=jnp.float32)
              + winb_ref[...])
        eat = ea_ref[...]
        epw = epw_ref[...]
        ea = (eat[:, 0:1] * epw[0:1, :] + eat[:, 1:2] * epw[1:2, :]
              + eat[:, 2:3] * epw[2:3, :]) + epb_ref[...]
        rolled = jnp.concatenate([p3[:, 1:, :], p3[:, :1, :]], axis=1)
        psrc = jnp.concatenate([p3, p3], axis=1).reshape(bt * NE, DP)
        pdst = jnp.concatenate([rolled, rolled], axis=1).reshape(bt * NE, DP)
        eeb = jnp.broadcast_to(ee_ref[...], (bt * NE, DE))
        lhs_e = jnp.concatenate([ea, psrc, pdst, eeb],
                                axis=1).astype(jnp.bfloat16)
        pe = (jnp.dot(lhs_e, win, preferred_element_type=jnp.float32)
              + winb_ref[...])
        gt3 = jnp.broadcast_to(gt_ref[...].reshape(1, 1, D), (bt, 1, D))
        toks = jnp.concatenate(
            [gt3, pv.reshape(bt, NN, D), pe.reshape(bt, NE, D),
             jnp.zeros((bt, S - SV, D), jnp.float32)], axis=1)
        h_scr[...] = toks.reshape(R, D)

    x = h_scr[...]
    xb16 = x.astype(jnp.bfloat16)
    qkv = (jnp.dot(xb16, qw_ref[0], preferred_element_type=jnp.float32)
           + qb_ref[0])
    qkvb = qkv.astype(jnp.bfloat16)

    bias = jnp.where(
        jax.lax.broadcasted_iota(jnp.int32, (S, S), 1) < SV, 0.0, -1e9)

    for c in range(bt):
        r0 = c * S
        qg = qkvb[r0:r0 + S, :]
        for h in range(H):
            q = qg[:, h * DH:(h + 1) * DH]
            k = qg[:, D + h * DH:D + (h + 1) * DH]
            v = qg[:, 2 * D + h * DH:2 * D + (h + 1) * DH]
            s = jax.lax.dot_general(q, k, (((1,), (1,)), ((), ())),
                                    preferred_element_type=jnp.float32)
            p = jnp.exp(s + bias)
            p = p * pl.reciprocal(jnp.sum(p, axis=-1, keepdims=True),
                                  approx=True)
            ctx = jnp.dot(p.astype(jnp.bfloat16), v,
                          preferred_element_type=jnp.float32)
            ctx_scr[r0:r0 + S, h * DH:(h + 1) * DH] = ctx

    attn = (jnp.dot(ctx_scr[...].astype(jnp.bfloat16), aw_ref[0],
                    preferred_element_type=jnp.float32) + ab_ref[0])
    h1 = _layernorm(attn + x, g1_ref[0], b1_ref[0], EPS)

    inter = (jnp.dot(h1.astype(jnp.bfloat16), iw_ref[0],
                     preferred_element_type=jnp.float32) + ib_ref[0])
    inter = _gelu_tanh(inter)
    ffn = (jnp.dot(inter.astype(jnp.bfloat16), ow_ref[0],
                   preferred_element_type=jnp.float32) + ob_ref[0])
    h2 = _layernorm(ffn + h1, g2_ref[0], b2_ref[0], EPS)
    h_scr[...] = h2

    @pl.when(l == pl.num_programs(1) - 1)
    def _():
        o_ref[...] = h2.reshape(bt, S, D)[:, :SV, :].astype(o_ref.dtype)


def _encoder(x, edge_attr, P3, edge_proj_w, edge_proj_b, E_V, E_E,
             w_in_w, w_in_b, graph_token, stk, *, bt=8):
    kern = functools.partial(_enc_kernel, bt=bt)

    def wspec(shape):
        n = len(shape)
        return pl.BlockSpec((1,) + shape, lambda b, l: (l,) + (0,) * n)

    def cspec(shape):
        n = len(shape)
        return pl.BlockSpec(shape, lambda b, l: (0,) * n)

    return pl.pallas_call(
        kern,
        out_shape=jax.ShapeDtypeStruct((G, SV, D), jnp.float32),
        grid=(G // bt, L),
        in_specs=[
            pl.BlockSpec((bt * NN, FD), lambda b, l: (b, 0)),
            pl.BlockSpec((bt * NE, 3), lambda b, l: (b, 0)),
            pl.BlockSpec((bt, NN, DP), lambda b, l: (b, 0, 0)),
            cspec((3, FD)), cspec((1, FD)),
            cspec((1, DE)), cspec((1, DE)),
            cspec((FD + 2 * DP + DE, D)), cspec((1, D)),
            cspec((1, D)),
            wspec((D, 3 * D)), wspec((1, 3 * D)),
            wspec((D, D)), wspec((1, D)),
            wspec((1, D)), wspec((1, D)),
            wspec((D, 2 * D)), wspec((1, 2 * D)),
            wspec((2 * D, D)), wspec((1, D)),
            wspec((1, D)), wspec((1, D)),
        ],
        out_specs=pl.BlockSpec((bt, SV, D), lambda b, l: (b, 0, 0)),
        scratch_shapes=[pltpu.VMEM((bt * S, D), jnp.float32),
                        pltpu.VMEM((bt * S, D), jnp.float32)],
        compiler_params=pltpu.CompilerParams(
            dimension_semantics=("parallel", "arbitrary"),
            vmem_limit_bytes=50 * 1024 * 1024),
    )(x, edge_attr, P3,
      edge_proj_w, edge_proj_b, E_V, E_E, w_in_w, w_in_b, graph_token,
      stk["qkv_w"], stk["qkv_b"], stk["ao_w"], stk["ao_b"],
      stk["ln1_g"], stk["ln1_b"], stk["i_w"], stk["i_b"],
      stk["o_w"], stk["o_b"], stk["ln2_g"], stk["ln2_b"])


def kernel(E_V, E_E, edge_proj_w, edge_proj_b, w_in_w, w_in_b, graph_token,
           enc_qkv_w, enc_qkv_b, enc_ao_w, enc_ao_b, enc_ln1_g, enc_ln1_b,
           enc_i_w, enc_i_b, enc_o_w, enc_o_b, enc_ln2_g, enc_ln2_b,
           x, edge_attr, id_key):
    kg_stack = _split_chain(id_key)
    gm = jax.vmap(lambda k: jax.random.normal(k, (NN, NN), jnp.float32))(
        kg_stack)
    q_orf = _qr_batched(gm)
    P3 = jnp.pad(q_orf, ((0, 0), (0, 0), (0, DP - NN)))

    stk = {"qkv_w": enc_qkv_w, "qkv_b": enc_qkv_b,
           "ao_w": enc_ao_w, "ao_b": enc_ao_b,
           "ln1_g": enc_ln1_g, "ln1_b": enc_ln1_b,
           "i_w": enc_i_w, "i_b": enc_i_b,
           "o_w": enc_o_w, "o_b": enc_o_b,
           "ln2_g": enc_ln2_g, "ln2_b": enc_ln2_b}
    h = _encoder(x, edge_attr, P3, edge_proj_w, edge_proj_b, E_V, E_E,
                 w_in_w, w_in_b, graph_token, stk)

    masks = jnp.ones((G, SV), dtype=bool)
    return h, masks
